# speedup vs baseline: 6.4083x; 6.4083x over previous
import jax
import jax.numpy as jnp
from jax import lax
from jax.experimental import pallas as pl
from jax.experimental.pallas import tpu as pltpu
from jax.experimental.pallas import tpu_sc as plsc

NC = 2
NS = 16
NW = NC * NS
BATCH = 1024
SEQ_LEN = 200
EMBED_DIM = 128
ROWS = BATCH * SEQ_LEN
ROWS_PER_W = ROWS // NW
CHUNK = SEQ_LEN
NCHUNK = ROWS_PER_W // CHUNK
IDX_MINOR = 100
IDX_ROWS_PER_W = ROWS_PER_W // IDX_MINOR
NGRP = EMBED_DIM // 16


def _sc_body(idx_hbm, tok_hbm, pos_hbm, out_hbm,
             idx_v, pos_v, buf0, buf1, gsem0, gsem1, osem0, osem1):
    wid = lax.axis_index("s") * NC + lax.axis_index("c")
    base_row = wid * ROWS_PER_W
    idx_base = wid * IDX_ROWS_PER_W

    bufs = (buf0, buf1)
    gsems = (gsem0, gsem1)
    osems = (osem0, osem1)

    pltpu.sync_copy(pos_hbm, pos_v)
    pltpu.sync_copy(idx_hbm.at[pl.ds(idx_base, IDX_ROWS_PER_W)], idx_v)

    def issue_gather(c, b):
        pltpu.async_copy(tok_hbm.at[idx_v.at[2 * c]],
                         bufs[b].at[pl.ds(0, IDX_MINOR)], gsems[b])
        pltpu.async_copy(tok_hbm.at[idx_v.at[2 * c + 1]],
                         bufs[b].at[pl.ds(IDX_MINOR, IDX_MINOR)], gsems[b])

    def wait_gather(b):
        pltpu.make_async_copy(tok_hbm.at[pl.ds(0, CHUNK)], bufs[b],
                              gsems[b]).wait()

    def wait_out(b):
        pltpu.make_async_copy(bufs[b], out_hbm.at[pl.ds(0, CHUNK)],
                              osems[b]).wait()

    def add_pos(b):
        buf = bufs[b]

        @pl.loop(0, CHUNK, unroll=2)
        def _(r):
            for g in range(NGRP):
                x = pos_v[r, pl.ds(g * 16, 16)]
                plsc.addupdate(buf.at[r, pl.ds(g * 16, 16)], x)

    issue_gather(0, 0)

    @pl.loop(0, NCHUNK // 2)
    def _(p):
        for b in range(2):
            c = p * 2 + b
            nb = 1 - b
            if b == 0:
                @pl.when(p > 0)
                def _():
                    wait_out(nb)

                issue_gather(c + 1, nb)
            else:
                @pl.when(p < NCHUNK // 2 - 1)
                def _():
                    wait_out(nb)
                    issue_gather(c + 1, nb)

            wait_gather(b)
            add_pos(b)
            pltpu.async_copy(bufs[b],
                             out_hbm.at[pl.ds(base_row + c * CHUNK, CHUNK)],
                             osems[b])

    wait_out(0)
    wait_out(1)


def kernel(inputs, token_table, pos_table):
    b, l = inputs.shape
    idx = inputs.reshape(-1, IDX_MINOR).astype(jnp.int32)
    mesh = plsc.VectorSubcoreMesh(core_axis_name="c", subcore_axis_name="s")
    out = pl.kernel(
        _sc_body,
        out_type=jax.ShapeDtypeStruct((ROWS, EMBED_DIM), jnp.float32),
        mesh=mesh,
        scratch_types=[
            pltpu.VMEM((IDX_ROWS_PER_W, IDX_MINOR), jnp.int32),
            pltpu.VMEM((SEQ_LEN, EMBED_DIM), jnp.float32),
            pltpu.VMEM((CHUNK, EMBED_DIM), jnp.float32),
            pltpu.VMEM((CHUNK, EMBED_DIM), jnp.float32),
            pltpu.SemaphoreType.DMA,
            pltpu.SemaphoreType.DMA,
            pltpu.SemaphoreType.DMA,
            pltpu.SemaphoreType.DMA,
        ],
    )(idx, token_table, pos_table)
    return out.reshape(b, l, EMBED_DIM)
